# Initial kernel scaffold; baseline (speedup 1.0000x reference)
import jax
import jax.numpy as jnp
from jax.experimental import pallas as pl


def kernel(x, edge_index, edge_weight, W1, b1, gamma, beta, Wg, att_src, att_dst, bg, W2, b2, W3, b3):
    raise NotImplementedError("write your pallas kernel here")


# speedup vs baseline: 16.0320x; 16.0320x over previous
import functools

import numpy as np

import jax
import jax.numpy as jnp
from jax import lax
from jax.experimental import pallas as pl
from jax.experimental.pallas import tpu as pltpu
from jax.experimental.pallas import tpu_sc as plsc

N = 10000
E = 320000
D = 128
ET = E + N
NW = 32
K = 128
C = -(-ET // (NW * K))
EPW = C * K
ETP = NW * EPW
RPT = 632
NP = 16 * RPT

_mesh = plsc.VectorSubcoreMesh(core_axis_name="c", subcore_axis_name="s")
_sc_params = pltpu.CompilerParams(needs_layout_passes=False)


def _f16(v, dtype=jnp.int32):
    return lax.full((16,), v, dtype)


def _splat(vec16, j):
    oh = jnp.where(lax.iota(jnp.int32, 16) == j, vec16, 0.0)
    return lax.full((16,), jnp.sum(oh), jnp.float32)


@functools.partial(
    pl.kernel,
    out_type=jax.ShapeDtypeStruct((2 * NP,), jnp.float32),
    mesh=_mesh,
    compiler_params=_sc_params,
    scratch_types=[
        pltpu.VMEM((C, K), jnp.int32),
        pltpu.VMEM((C, K), jnp.float32),
        pltpu.VMEM((RPT,), jnp.float32),
        pltpu.VMEM_SHARED((NP,), jnp.float32),
    ],
)
def _deg_sc(dst_hbm, ew_hbm, z1_hbm, out_hbm, dst_v, ew_v, buf_v, acc_sh):
    cid = lax.axis_index("c")
    sid = lax.axis_index("s")
    wid = sid * 2 + cid
    pltpu.sync_copy(z1_hbm.at[pl.ds(0, RPT)], buf_v)
    pltpu.sync_copy(buf_v, acc_sh.at[pl.ds(sid * RPT, RPT)])
    pltpu.sync_copy(dst_hbm.at[wid], dst_v)
    pltpu.sync_copy(ew_hbm.at[wid], ew_v)
    plsc.subcore_barrier()

    def body(c, _):
        pltpu.sync_copy(ew_v.at[c], acc_sh.at[dst_v.at[c]], add=True)
        return 0

    lax.fori_loop(0, C, body, 0)
    plsc.subcore_barrier()
    pltpu.sync_copy(acc_sh.at[pl.ds(sid * RPT, RPT)], buf_v)
    pltpu.sync_copy(buf_v, out_hbm.at[pl.ds(cid * NP + sid * RPT, RPT)])


@functools.partial(
    pl.kernel,
    out_type=jax.ShapeDtypeStruct((2, NP, D), jnp.float32),
    mesh=_mesh,
    compiler_params=_sc_params,
    scratch_types=[
        pltpu.VMEM((C, K), jnp.int32),
        pltpu.VMEM((C, K), jnp.int32),
        pltpu.VMEM((K,), jnp.float32),
        pltpu.VMEM((K, D), jnp.float32),
        pltpu.VMEM_SHARED((NP, D), jnp.float32),
        pltpu.SemaphoreType.DMA,
    ],
)
def _mp_sc(hs_hbm, src_hbm, dst_hbm, w_hbm, z_hbm, out_hbm,
           src_v, dst_v, wrow_v, rows_v, acc_sh, sem):
    cid = lax.axis_index("c")
    sid = lax.axis_index("s")
    wid = sid * 2 + cid
    pltpu.sync_copy(z_hbm.at[pl.ds(0, K)], rows_v)
    for off, sz in ((0, K), (K, K), (2 * K, K), (3 * K, K), (4 * K, RPT - 4 * K)):
        pltpu.sync_copy(rows_v.at[pl.ds(0, sz)],
                        acc_sh.at[pl.ds(sid * RPT + off, sz)])
    pltpu.sync_copy(src_hbm.at[wid], src_v)
    pltpu.sync_copy(dst_hbm.at[wid], dst_v)
    plsc.subcore_barrier()

    def body(c, _):
        cp = pltpu.async_copy(hs_hbm.at[src_v.at[c]], rows_v, sem)
        pltpu.sync_copy(w_hbm.at[pl.ds(wid * EPW + c * K, K)], wrow_v)
        cp.wait()
        for j16 in range(K // 16):
            wv = wrow_v[pl.ds(j16 * 16, 16)]
            for j in range(16):
                jj = j16 * 16 + j
                ws = _splat(wv, j)
                for v in range(D // 16):
                    sl = pl.ds(v * 16, 16)
                    rows_v[jj, sl] = rows_v[jj, sl] * ws
        pltpu.sync_copy(rows_v, acc_sh.at[dst_v.at[c]], add=True)
        return 0

    lax.fori_loop(0, C, body, 0)
    plsc.subcore_barrier()
    for off, sz in ((0, K), (K, K), (2 * K, K), (3 * K, K), (4 * K, RPT - 4 * K)):
        pltpu.sync_copy(acc_sh.at[pl.ds(sid * RPT + off, sz)],
                        rows_v.at[pl.ds(0, sz)])
        pltpu.sync_copy(rows_v.at[pl.ds(0, sz)],
                        out_hbm.at[cid, pl.ds(sid * RPT + off, sz)])


@functools.partial(
    pl.kernel,
    out_type=(
        jax.ShapeDtypeStruct((NW, C, K), jnp.float32),
        jax.ShapeDtypeStruct((2 * NP,), jnp.float32),
    ),
    mesh=_mesh,
    compiler_params=_sc_params,
    scratch_types=[
        pltpu.VMEM((C, K), jnp.int32),
        pltpu.VMEM((C, K), jnp.int32),
        pltpu.VMEM((C, K), jnp.float32),
        pltpu.VMEM((N,), jnp.float32),
        pltpu.VMEM((N,), jnp.float32),
        pltpu.VMEM((N,), jnp.float32),
        pltpu.VMEM((C, K), jnp.float32),
        pltpu.VMEM((RPT,), jnp.float32),
        pltpu.VMEM_SHARED((NP,), jnp.float32),
    ],
)
def _gat_a_sc(src_hbm, dst_hbm, valid_hbm, as_hbm, ad_hbm, b_hbm, z1_hbm,
              ex_hbm, den_hbm,
              src_v, dst_v, valid_v, as_v, ad_v, b_v, ex_v, buf_v, acc_sh):
    cid = lax.axis_index("c")
    sid = lax.axis_index("s")
    wid = sid * 2 + cid
    pltpu.sync_copy(z1_hbm.at[pl.ds(0, RPT)], buf_v)
    pltpu.sync_copy(buf_v, acc_sh.at[pl.ds(sid * RPT, RPT)])
    pltpu.sync_copy(src_hbm.at[wid], src_v)
    pltpu.sync_copy(dst_hbm.at[wid], dst_v)
    pltpu.sync_copy(valid_hbm.at[wid], valid_v)
    pltpu.sync_copy(as_hbm, as_v)
    pltpu.sync_copy(ad_hbm, ad_v)
    pltpu.sync_copy(b_hbm, b_v)
    plsc.subcore_barrier()

    def body(c, _):
        for j16 in range(K // 16):
            sl = pl.ds(j16 * 16, 16)
            srcv = src_v[c, sl]
            dstv = dst_v[c, sl]
            a = plsc.load_gather(as_v, [srcv]) + plsc.load_gather(ad_v, [dstv])
            alpha = jnp.maximum(a, a * 0.2)
            bv = plsc.load_gather(b_v, [dstv])
            ex = jnp.exp(alpha - bv) * valid_v[c, sl]
            ex_v[c, sl] = ex
        pltpu.sync_copy(ex_v.at[c], acc_sh.at[dst_v.at[c]], add=True)
        return 0

    lax.fori_loop(0, C, body, 0)
    plsc.subcore_barrier()
    pltpu.sync_copy(acc_sh.at[pl.ds(sid * RPT, RPT)], buf_v)
    pltpu.sync_copy(buf_v, den_hbm.at[pl.ds(cid * NP + sid * RPT, RPT)])
    pltpu.sync_copy(ex_v, ex_hbm.at[wid])


@functools.partial(
    pl.kernel,
    out_type=jax.ShapeDtypeStruct((2, NP, D), jnp.float32),
    mesh=_mesh,
    compiler_params=_sc_params,
    scratch_types=[
        pltpu.VMEM((C, K), jnp.int32),
        pltpu.VMEM((C, K), jnp.int32),
        pltpu.VMEM((N,), jnp.float32),
        pltpu.VMEM((K,), jnp.float32),
        pltpu.VMEM((K, D), jnp.float32),
        pltpu.VMEM_SHARED((NP, D), jnp.float32),
        pltpu.SemaphoreType.DMA,
    ],
)
def _gat_b_sc(h2_hbm, src_hbm, dst_hbm, ex_hbm, den_hbm, z_hbm, out_hbm,
              src_v, dst_v, den_v, exrow_v, rows_v, acc_sh, sem):
    cid = lax.axis_index("c")
    sid = lax.axis_index("s")
    wid = sid * 2 + cid
    pltpu.sync_copy(z_hbm.at[pl.ds(0, K)], rows_v)
    for off, sz in ((0, K), (K, K), (2 * K, K), (3 * K, K), (4 * K, RPT - 4 * K)):
        pltpu.sync_copy(rows_v.at[pl.ds(0, sz)],
                        acc_sh.at[pl.ds(sid * RPT + off, sz)])
    pltpu.sync_copy(src_hbm.at[wid], src_v)
    pltpu.sync_copy(dst_hbm.at[wid], dst_v)
    pltpu.sync_copy(den_hbm, den_v)
    plsc.subcore_barrier()

    def body(c, _):
        cp = pltpu.async_copy(h2_hbm.at[src_v.at[c]], rows_v, sem)
        pltpu.sync_copy(ex_hbm.at[pl.ds(wid * EPW + c * K, K)], exrow_v)
        cp.wait()
        for j16 in range(K // 16):
            sl = pl.ds(j16 * 16, 16)
            dstv = dst_v[c, sl]
            deng = plsc.load_gather(den_v, [dstv])
            c16 = exrow_v[sl] / deng
            for j in range(16):
                jj = j16 * 16 + j
                cs = _splat(c16, j)
                for v in range(D // 16):
                    slv = pl.ds(v * 16, 16)
                    rows_v[jj, slv] = rows_v[jj, slv] * cs
        pltpu.sync_copy(rows_v, acc_sh.at[dst_v.at[c]], add=True)
        return 0

    lax.fori_loop(0, C, body, 0)
    plsc.subcore_barrier()
    for off, sz in ((0, K), (K, K), (2 * K, K), (3 * K, K), (4 * K, RPT - 4 * K)):
        pltpu.sync_copy(acc_sh.at[pl.ds(sid * RPT + off, sz)],
                        rows_v.at[pl.ds(0, sz)])
        pltpu.sync_copy(rows_v.at[pl.ds(0, sz)],
                        out_hbm.at[cid, pl.ds(sid * RPT + off, sz)])


def _tc1_body(x_ref, degp_ref, w1_ref, hs_ref, dinv_ref):
    deg = (degp_ref[0] + degp_ref[1])[:N]
    dinv = jnp.where(deg > 0, lax.rsqrt(deg), 0.0)
    xx = x_ref[...]
    xx = jnp.where(jnp.isnan(xx) | jnp.isinf(xx), jnp.zeros_like(xx), xx)
    h = jnp.dot(xx, w1_ref[...], preferred_element_type=jnp.float32)
    hs_ref[...] = h * dinv[:, None]
    dinv_ref[...] = dinv


def _tc2_body(aggp_ref, dinv_ref, b1_ref, gamma_ref, beta_ref, wg_ref,
              asrc_ref, adst_ref, h2_ref, as_ref, ad_ref, b_ref):
    dinv = dinv_ref[...]
    z = (aggp_ref[0, :N] + aggp_ref[1, :N]) * dinv[:, None] + b1_ref[...][None, :]
    mean = jnp.mean(z, axis=0)
    zc = z - mean[None, :]
    var = jnp.mean(zc * zc, axis=0)
    xb = zc * lax.rsqrt(var + 1e-5)[None, :] * gamma_ref[...][None, :] + beta_ref[...][None, :]
    xr = jnp.maximum(xb, 0.0)
    h2 = jnp.dot(xr, wg_ref[...], preferred_element_type=jnp.float32)
    a_s = jnp.sum(h2 * asrc_ref[...][None, :], axis=1)
    a_d = jnp.sum(h2 * adst_ref[...][None, :], axis=1)
    t = jnp.max(a_s) + a_d
    h2_ref[...] = h2
    as_ref[...] = a_s
    ad_ref[...] = a_d
    b_ref[...] = jnp.maximum(t, t * 0.2)


def _tcden_body(denp_ref, den_ref):
    den_ref[...] = (denp_ref[0] + denp_ref[1])[:N]


def _tc3_body(aggp_ref, bg_ref, dinv_ref, w2_ref, hs_ref):
    x2 = jnp.maximum(aggp_ref[0, :N] + aggp_ref[1, :N] + bg_ref[...][None, :], 0.0)
    h3 = jnp.dot(x2, w2_ref[...], preferred_element_type=jnp.float32)
    hs_ref[...] = h3 * dinv_ref[...][:, None]


def _tc4_body(aggp_ref, dinv_ref, b2_ref, w3_ref, hs_ref):
    dinv = dinv_ref[...]
    x3 = (aggp_ref[0, :N] + aggp_ref[1, :N]) * dinv[:, None] + b2_ref[...][None, :]
    h4 = jnp.dot(x3, w3_ref[...], preferred_element_type=jnp.float32)
    hs_ref[...] = h4 * dinv[:, None]


def _tc5_body(aggp_ref, dinv_ref, b3_ref, out_ref):
    out_ref[...] = (aggp_ref[0, :N] + aggp_ref[1, :N]) * dinv_ref[...][:, None] + b3_ref[...][None, :]


def _tc(body, out_shape):
    return pl.pallas_call(body, out_shape=out_shape)


_F = jnp.float32


def kernel(x, edge_index, edge_weight, W1, b1, gamma, beta, Wg, att_src,
           att_dst, bg, W2, b2, W3, b3):
    loop = jnp.arange(N, dtype=jnp.int32)
    src = jnp.concatenate([edge_index[0].astype(jnp.int32), loop])
    dst = jnp.concatenate([edge_index[1].astype(jnp.int32), loop])
    ew = jnp.concatenate([edge_weight.astype(_F), jnp.ones((N,), _F)])
    pad = ETP - ET
    srcp = jnp.pad(src, (0, pad)).reshape(NW, C, K)
    dstp = jnp.pad(dst, (0, pad)).reshape(NW, C, K)
    ewp = jnp.pad(ew, (0, pad)).reshape(NW, C, K)
    ewf = ewp.reshape(ETP)
    validp = jnp.pad(jnp.ones((ET,), _F), (0, pad)).reshape(NW, C, K)
    z1 = jnp.zeros((NP,), _F)
    zrows = jnp.zeros((NP, D), _F)

    degp = _deg_sc(dstp, ewp, z1).reshape(2, NP)
    hs1, dinv = _tc(_tc1_body, (jax.ShapeDtypeStruct((N, D), _F),
                                jax.ShapeDtypeStruct((N,), _F)))(x, degp, W1)
    agg1p = _mp_sc(hs1, srcp, dstp, ewf, zrows)
    h2, a_s, a_d, bnd = _tc(_tc2_body, (jax.ShapeDtypeStruct((N, D), _F),
                                        jax.ShapeDtypeStruct((N,), _F),
                                        jax.ShapeDtypeStruct((N,), _F),
                                        jax.ShapeDtypeStruct((N,), _F)))(
        agg1p, dinv, b1, gamma, beta, Wg, att_src, att_dst)
    exp_, denp = _gat_a_sc(srcp, dstp, validp, a_s, a_d, bnd, z1)
    denp = denp.reshape(2, NP)
    den = _tc(_tcden_body, jax.ShapeDtypeStruct((N,), _F))(denp)
    agg2p = _gat_b_sc(h2, srcp, dstp, exp_.reshape(ETP), den, zrows)
    hs3 = _tc(_tc3_body, jax.ShapeDtypeStruct((N, D), _F))(agg2p, bg, dinv, W2)
    agg3p = _mp_sc(hs3, srcp, dstp, ewf, zrows)
    hs4 = _tc(_tc4_body, jax.ShapeDtypeStruct((N, D), _F))(agg3p, dinv, b2, W3)
    agg4p = _mp_sc(hs4, srcp, dstp, ewf, zrows)
    out = _tc(_tc5_body, jax.ShapeDtypeStruct((N, D), _F))(agg4p, dinv, b3)
    return out
